# Initial kernel scaffold; baseline (speedup 1.0000x reference)
import jax
import jax.numpy as jnp
from jax.experimental import pallas as pl


def kernel(input_ids, token_type_ids, tok_emb, pos_emb, type_emb, gamma, beta):
    raise NotImplementedError("write your pallas kernel here")


# speedup vs baseline: 1.9889x; 1.9889x over previous
import functools

import jax
import jax.numpy as jnp
from jax import lax
from jax.experimental import pallas as pl
from jax.experimental.pallas import tpu as pltpu
from jax.experimental.pallas import tpu_sc as plsc

NC, NS, L = 2, 16, 16
NW = NC * NS
BATCH, SEQ, HIDDEN = 1024, 512, 128
NTOK = BATCH * SEQ
TPW = NTOK // NW
C = 128
NCHUNK = TPW // C
NJ = HIDDEN // L


def _rsqrt16(v):
    i = lax.bitcast_convert_type(v, jnp.int32)
    y = lax.bitcast_convert_type(jnp.int32(0x5F3759DF) - (i >> 1), jnp.float32)
    for _ in range(3):
        y = y * (1.5 - 0.5 * v * y * y)
    return y


def _body(ids_hbm, tt_hbm, tok_hbm, pos_hbm, type_hbm, gam_hbm, bet_hbm,
          out_hbm, idx_v, ttv, rows_v, out_v, pos_v, type_v, gam_v, bet_v,
          sem):
    wid = lax.axis_index("s") * NC + lax.axis_index("c")

    pltpu.sync_copy(pos_hbm, pos_v)
    pltpu.sync_copy(type_hbm, type_v)
    pltpu.sync_copy(gam_hbm, gam_v)
    pltpu.sync_copy(bet_hbm, bet_v)

    @pl.loop(0, NCHUNK)
    def _chunk(ci):
        base = wid * TPW + ci * C
        p0 = (ci % (SEQ // C)) * C

        pltpu.sync_copy(ids_hbm.at[pl.ds(base, C)], idx_v)
        pltpu.sync_copy(tt_hbm.at[pl.ds(base, C)], ttv.at[pl.ds(0, C)])
        pltpu.async_copy(tok_hbm.at[idx_v], rows_v, sem).wait()

        @pl.loop(0, C)
        def _tok(i):
            tt = ttv[pl.ds(i, L)][0]
            p = p0 + i
            xs = []
            for j in range(NJ):
                sl = pl.ds(j * L, L)
                xs.append(rows_v[i, sl] + pos_v[p, sl] + type_v[tt, sl])
            s = xs[0]
            sq = xs[0] * xs[0]
            for j in range(1, NJ):
                s = s + xs[j]
                sq = sq + xs[j] * xs[j]
            tot = jnp.sum(s)
            totsq = jnp.sum(sq)
            meanv = lax.broadcast(tot, (L,)) * (1.0 / HIDDEN)
            varv = lax.broadcast(totsq, (L,)) * (1.0 / HIDDEN) \
                - meanv * meanv + 1e-5
            rstd = _rsqrt16(varv)
            for j in range(NJ):
                sl = pl.ds(j * L, L)
                out_v[i, sl] = (xs[j] - meanv) * rstd * gam_v[sl] + bet_v[sl]

        pltpu.sync_copy(out_v, out_hbm.at[pl.ds(base, C)])


@jax.jit
def _run(ids, tts, tok_emb, pos_emb, type_emb, gamma, beta):
    mesh = plsc.VectorSubcoreMesh(core_axis_name="c", subcore_axis_name="s",
                                  num_cores=NC, num_subcores=NS)
    f = pl.kernel(
        _body,
        out_type=jax.ShapeDtypeStruct((NTOK, HIDDEN), jnp.float32),
        mesh=mesh,
        compiler_params=pltpu.CompilerParams(needs_layout_passes=False),
        scratch_types=[
            pltpu.VMEM((C,), jnp.int32),
            pltpu.VMEM((C + L,), jnp.int32),
            pltpu.VMEM((C, HIDDEN), jnp.float32),
            pltpu.VMEM((C, HIDDEN), jnp.float32),
            pltpu.VMEM((SEQ, HIDDEN), jnp.float32),
            pltpu.VMEM((3, HIDDEN), jnp.float32),
            pltpu.VMEM((HIDDEN,), jnp.float32),
            pltpu.VMEM((HIDDEN,), jnp.float32),
            pltpu.SemaphoreType.DMA,
        ],
    )
    return f(ids, tts, tok_emb, pos_emb, type_emb, gamma, beta)


def kernel(input_ids, token_type_ids, tok_emb, pos_emb, type_emb, gamma, beta):
    ids = input_ids.reshape(-1).astype(jnp.int32)
    tts = token_type_ids.reshape(-1).astype(jnp.int32)
    out = _run(ids, tts, tok_emb, pos_emb, type_emb, gamma, beta)
    return out.reshape(BATCH, SEQ, HIDDEN)


# speedup vs baseline: 2.4660x; 1.2399x over previous
import functools

import jax
import jax.numpy as jnp
from jax import lax
from jax.experimental import pallas as pl
from jax.experimental.pallas import tpu as pltpu
from jax.experimental.pallas import tpu_sc as plsc

NC, NS, L = 2, 16, 16
NW = NC * NS
BATCH, SEQ, HIDDEN = 1024, 512, 128
NTOK = BATCH * SEQ
TPW = NTOK // NW
C = 64
CPSS = 32
SCTOK = C * CPSS
NSS = TPW // SCTOK
NJ = HIDDEN // L


def _rsqrt16(v):
    i = lax.bitcast_convert_type(v, jnp.int32)
    y = lax.bitcast_convert_type(jnp.int32(0x5F3759DF) - (i >> 1), jnp.float32)
    for _ in range(3):
        y = y * (1.5 - 0.5 * v * y * y)
    return y


def _body(ids_hbm, tt_hbm, tok_hbm, pos_hbm, type_hbm, gam_hbm, bet_hbm,
          out_hbm, idx_sc, tt_sc, rows0, rows1, out0, out1, pos_v, type_v,
          gam_v, bet_v, sem_g, sem_o):
    rows = (rows0, rows1)
    outs = (out0, out1)
    wid = lax.axis_index("s") * NC + lax.axis_index("c")

    pltpu.sync_copy(pos_hbm, pos_v)
    pltpu.sync_copy(type_hbm, type_v)
    pltpu.sync_copy(gam_hbm, gam_v)
    pltpu.sync_copy(bet_hbm, bet_v)

    def gather_start(k, buf):
        pltpu.async_copy(tok_hbm.at[idx_sc.at[pl.ds(k * C, C)]], buf, sem_g)

    def gather_wait(buf):
        pltpu.make_async_copy(
            tok_hbm.at[idx_sc.at[pl.ds(0, C)]], buf, sem_g).wait()

    def out_wait(buf):
        pltpu.make_async_copy(buf, out_hbm.at[pl.ds(0, C)], sem_o).wait()

    def compute(p0, toff, rbuf, obuf):
        @pl.loop(0, C, unroll=2)
        def _tok(i):
            tt = tt_sc[pl.ds(toff + i, L)][0]
            p = p0 + i
            xs = []
            for j in range(NJ):
                sl = pl.ds(j * L, L)
                xs.append(rbuf[i, sl] + pos_v[p, sl] + type_v[tt, sl])
            s = xs[0]
            sq = xs[0] * xs[0]
            for j in range(1, NJ):
                s = s + xs[j]
                sq = sq + xs[j] * xs[j]
            tot = jnp.sum(s)
            totsq = jnp.sum(sq)
            meanv = lax.broadcast(tot, (L,)) * (1.0 / HIDDEN)
            varv = lax.broadcast(totsq, (L,)) * (1.0 / HIDDEN) \
                - meanv * meanv + 1e-5
            rstd = _rsqrt16(varv)
            for j in range(NJ):
                sl = pl.ds(j * L, L)
                obuf[i, sl] = (xs[j] - meanv) * rstd * gam_v[sl] + bet_v[sl]

    @pl.loop(0, NSS)
    def _ss(s):
        ids_off = wid * TPW + s * SCTOK
        pltpu.sync_copy(ids_hbm.at[pl.ds(ids_off, SCTOK)], idx_sc)
        pltpu.sync_copy(tt_hbm.at[pl.ds(ids_off, SCTOK)],
                        tt_sc.at[pl.ds(0, SCTOK)])
        gather_start(0, rows[0])

        @pl.loop(0, CPSS, step=2)
        def _chunk(k0):
            for b in range(2):
                k = k0 + b
                gci = s * CPSS + k
                gather_wait(rows[b])

                @pl.when(k + 1 < CPSS)
                def _():
                    gather_start(k + 1, rows[1 - b])

                @pl.when(gci >= 2)
                def _():
                    out_wait(outs[b])

                p0 = (gci % (SEQ // C)) * C
                compute(p0, k * C, rows[b], outs[b])
                pltpu.async_copy(
                    outs[b], out_hbm.at[pl.ds(wid * TPW + gci * C, C)],
                    sem_o)

    out_wait(outs[0])
    out_wait(outs[1])


@jax.jit
def _run(ids, tts, tok_emb, pos_emb, type_emb, gamma, beta):
    mesh = plsc.VectorSubcoreMesh(core_axis_name="c", subcore_axis_name="s",
                                  num_cores=NC, num_subcores=NS)
    f = pl.kernel(
        _body,
        out_type=jax.ShapeDtypeStruct((NTOK, HIDDEN), jnp.float32),
        mesh=mesh,
        compiler_params=pltpu.CompilerParams(needs_layout_passes=False),
        scratch_types=[
            pltpu.VMEM((SCTOK,), jnp.int32),
            pltpu.VMEM((SCTOK + L,), jnp.int32),
            pltpu.VMEM((C, HIDDEN), jnp.float32),
            pltpu.VMEM((C, HIDDEN), jnp.float32),
            pltpu.VMEM((C, HIDDEN), jnp.float32),
            pltpu.VMEM((C, HIDDEN), jnp.float32),
            pltpu.VMEM((SEQ, HIDDEN), jnp.float32),
            pltpu.VMEM((3, HIDDEN), jnp.float32),
            pltpu.VMEM((HIDDEN,), jnp.float32),
            pltpu.VMEM((HIDDEN,), jnp.float32),
            pltpu.SemaphoreType.DMA,
            pltpu.SemaphoreType.DMA,
        ],
    )
    return f(ids, tts, tok_emb, pos_emb, type_emb, gamma, beta)


def kernel(input_ids, token_type_ids, tok_emb, pos_emb, type_emb, gamma, beta):
    ids = input_ids.reshape(-1).astype(jnp.int32)
    tts = token_type_ids.reshape(-1).astype(jnp.int32)
    out = _run(ids, tts, tok_emb, pos_emb, type_emb, gamma, beta)
    return out.reshape(BATCH, SEQ, HIDDEN)


# speedup vs baseline: 2.5593x; 1.0378x over previous
import functools

import jax
import jax.numpy as jnp
from jax import lax
from jax.experimental import pallas as pl
from jax.experimental.pallas import tpu as pltpu
from jax.experimental.pallas import tpu_sc as plsc

NC, NS, L = 2, 16, 16
NW = NC * NS
BATCH, SEQ, HIDDEN = 1024, 512, 128
NTOK = BATCH * SEQ
TPW = NTOK // NW
C = 64
CPSS = 32
SCTOK = C * CPSS
NSS = TPW // SCTOK
NJ = HIDDEN // L


def _rsqrt16(v):
    i = lax.bitcast_convert_type(v, jnp.int32)
    y = lax.bitcast_convert_type(jnp.int32(0x5F3759DF) - (i >> 1), jnp.float32)
    for _ in range(2):
        y = y * (1.5 - 0.5 * v * y * y)
    return y


def _body(ids_hbm, tt_hbm, tok_hbm, pos_hbm, type_hbm, gam_hbm, bet_hbm,
          out_hbm, idx_sc, tt_sc, rows0, rows1, out0, out1, pos_v, type_v,
          gam_v, bet_v, sem_g, sem_o):
    rows = (rows0, rows1)
    outs = (out0, out1)
    wid = lax.axis_index("s") * NC + lax.axis_index("c")

    pltpu.sync_copy(pos_hbm, pos_v)
    pltpu.sync_copy(type_hbm, type_v)
    pltpu.sync_copy(gam_hbm, gam_v)
    pltpu.sync_copy(bet_hbm, bet_v)

    def gather_start(k, buf):
        pltpu.async_copy(tok_hbm.at[idx_sc.at[pl.ds(k * C, C)]], buf, sem_g)

    def gather_wait(buf):
        pltpu.make_async_copy(
            tok_hbm.at[idx_sc.at[pl.ds(0, C)]], buf, sem_g).wait()

    def out_wait(buf):
        pltpu.make_async_copy(buf, out_hbm.at[pl.ds(0, C)], sem_o).wait()

    def compute(p0, toff, rbuf, obuf):
        @pl.loop(0, C, unroll=4)
        def _tok(i):
            tt = tt_sc[pl.ds(toff + i, L)][0]
            p = p0 + i
            xs = []
            for j in range(NJ):
                sl = pl.ds(j * L, L)
                xs.append(rbuf[i, sl] + pos_v[p, sl] + type_v[tt, sl])

            def tree(vs):
                while len(vs) > 1:
                    vs = [a + b for a, b in zip(vs[::2], vs[1::2])]
                return vs[0]

            s = tree(xs)
            sq = tree([x * x for x in xs])
            tot = jnp.sum(s)
            totsq = jnp.sum(sq)
            meanv = lax.broadcast(tot, (L,)) * (1.0 / HIDDEN)
            varv = lax.broadcast(totsq, (L,)) * (1.0 / HIDDEN) \
                - meanv * meanv + 1e-5
            rstd = _rsqrt16(varv)
            for j in range(NJ):
                sl = pl.ds(j * L, L)
                obuf[i, sl] = (xs[j] - meanv) * rstd * gam_v[sl] + bet_v[sl]

    @pl.loop(0, NSS)
    def _ss(s):
        ids_off = wid * TPW + s * SCTOK
        pltpu.sync_copy(ids_hbm.at[pl.ds(ids_off, SCTOK)], idx_sc)
        pltpu.sync_copy(tt_hbm.at[pl.ds(ids_off, SCTOK)],
                        tt_sc.at[pl.ds(0, SCTOK)])
        gather_start(0, rows[0])

        @pl.loop(0, CPSS, step=2)
        def _chunk(k0):
            for b in range(2):
                k = k0 + b
                gci = s * CPSS + k
                gather_wait(rows[b])

                @pl.when(k + 1 < CPSS)
                def _():
                    gather_start(k + 1, rows[1 - b])

                @pl.when(gci >= 2)
                def _():
                    out_wait(outs[b])

                p0 = (gci % (SEQ // C)) * C
                compute(p0, k * C, rows[b], outs[b])
                pltpu.async_copy(
                    outs[b], out_hbm.at[pl.ds(wid * TPW + gci * C, C)],
                    sem_o)

    out_wait(outs[0])
    out_wait(outs[1])


@jax.jit
def _run(ids, tts, tok_emb, pos_emb, type_emb, gamma, beta):
    mesh = plsc.VectorSubcoreMesh(core_axis_name="c", subcore_axis_name="s",
                                  num_cores=NC, num_subcores=NS)
    f = pl.kernel(
        _body,
        out_type=jax.ShapeDtypeStruct((NTOK, HIDDEN), jnp.float32),
        mesh=mesh,
        compiler_params=pltpu.CompilerParams(needs_layout_passes=False),
        scratch_types=[
            pltpu.VMEM((SCTOK,), jnp.int32),
            pltpu.VMEM((SCTOK + L,), jnp.int32),
            pltpu.VMEM((C, HIDDEN), jnp.float32),
            pltpu.VMEM((C, HIDDEN), jnp.float32),
            pltpu.VMEM((C, HIDDEN), jnp.float32),
            pltpu.VMEM((C, HIDDEN), jnp.float32),
            pltpu.VMEM((SEQ, HIDDEN), jnp.float32),
            pltpu.VMEM((3, HIDDEN), jnp.float32),
            pltpu.VMEM((HIDDEN,), jnp.float32),
            pltpu.VMEM((HIDDEN,), jnp.float32),
            pltpu.SemaphoreType.DMA,
            pltpu.SemaphoreType.DMA,
        ],
    )
    return f(ids, tts, tok_emb, pos_emb, type_emb, gamma, beta)


def kernel(input_ids, token_type_ids, tok_emb, pos_emb, type_emb, gamma, beta):
    ids = input_ids.reshape(-1).astype(jnp.int32)
    tts = token_type_ids.reshape(-1).astype(jnp.int32)
    out = _run(ids, tts, tok_emb, pos_emb, type_emb, gamma, beta)
    return out.reshape(BATCH, SEQ, HIDDEN)


# speedup vs baseline: 3.6470x; 1.4250x over previous
import functools

import jax
import jax.numpy as jnp
from jax import lax
from jax.experimental import pallas as pl
from jax.experimental.pallas import tpu as pltpu
from jax.experimental.pallas import tpu_sc as plsc

NC, NS, L = 2, 16, 16
NW = NC * NS
BATCH, SEQ, HIDDEN = 1024, 512, 128
NTOK = BATCH * SEQ
TPW = NTOK // NW
C = 64
CPSS = 32
SCTOK = C * CPSS
NSS = TPW // SCTOK
NJ = HIDDEN // L


def _rsqrt16(v):
    i = lax.bitcast_convert_type(v, jnp.int32)
    y = lax.bitcast_convert_type(jnp.int32(0x5F3759DF) - (i >> 1), jnp.float32)
    for _ in range(2):
        y = y * (1.5 - 0.5 * v * y * y)
    return y


def _body(ids_hbm, tt_hbm, tok_hbm, pos_hbm, type_hbm, gam_hbm, bet_hbm,
          out_hbm, idx_sc, tt_sc, rows0, rows1, out0, out1, pos_v, type_v,
          gam_v, bet_v, sem_g, sem_o):
    rows = (rows0, rows1)
    outs = (out0, out1)
    wid = lax.axis_index("s") * NC + lax.axis_index("c")

    pltpu.sync_copy(pos_hbm, pos_v)
    pltpu.sync_copy(type_hbm, type_v)
    pltpu.sync_copy(gam_hbm, gam_v)
    pltpu.sync_copy(bet_hbm, bet_v)

    def gather_start(k, buf):
        pltpu.async_copy(tok_hbm.at[idx_sc.at[pl.ds(k * C, C)]], buf, sem_g)

    def gather_wait(buf):
        pltpu.make_async_copy(
            tok_hbm.at[idx_sc.at[pl.ds(0, C)]], buf, sem_g).wait()

    def out_wait(buf):
        pltpu.make_async_copy(buf, out_hbm.at[pl.ds(0, C)], sem_o).wait()

    def compute(p0, toff, rbuf, obuf):
        @plsc.parallel_loop(0, C, unroll=4)
        def _tok(i):
            tt = tt_sc[pl.ds(toff + i, L)][0]
            p = p0 + i
            xs = []
            for j in range(NJ):
                sl = pl.ds(j * L, L)
                xs.append(rbuf[i, sl] + pos_v[p, sl] + type_v[tt, sl])

            def tree(vs):
                while len(vs) > 1:
                    vs = [a + b for a, b in zip(vs[::2], vs[1::2])]
                return vs[0]

            s = tree(xs)
            sq = tree([x * x for x in xs])
            tot = jnp.sum(s)
            totsq = jnp.sum(sq)
            meanv = lax.broadcast(tot, (L,)) * (1.0 / HIDDEN)
            varv = lax.broadcast(totsq, (L,)) * (1.0 / HIDDEN) \
                - meanv * meanv + 1e-5
            rstd = _rsqrt16(varv)
            for j in range(NJ):
                sl = pl.ds(j * L, L)
                obuf[i, sl] = (xs[j] - meanv) * rstd * gam_v[sl] + bet_v[sl]

    @pl.loop(0, NSS)
    def _ss(s):
        ids_off = wid * TPW + s * SCTOK
        pltpu.sync_copy(ids_hbm.at[pl.ds(ids_off, SCTOK)], idx_sc)
        pltpu.sync_copy(tt_hbm.at[pl.ds(ids_off, SCTOK)],
                        tt_sc.at[pl.ds(0, SCTOK)])
        gather_start(0, rows[0])

        @pl.loop(0, CPSS, step=2)
        def _chunk(k0):
            for b in range(2):
                k = k0 + b
                gci = s * CPSS + k
                gather_wait(rows[b])

                @pl.when(k + 1 < CPSS)
                def _():
                    gather_start(k + 1, rows[1 - b])

                @pl.when(gci >= 2)
                def _():
                    out_wait(outs[b])

                p0 = (gci % (SEQ // C)) * C
                compute(p0, k * C, rows[b], outs[b])
                pltpu.async_copy(
                    outs[b], out_hbm.at[pl.ds(wid * TPW + gci * C, C)],
                    sem_o)

    out_wait(outs[0])
    out_wait(outs[1])


@jax.jit
def _run(ids, tts, tok_emb, pos_emb, type_emb, gamma, beta):
    mesh = plsc.VectorSubcoreMesh(core_axis_name="c", subcore_axis_name="s",
                                  num_cores=NC, num_subcores=NS)
    f = pl.kernel(
        _body,
        out_type=jax.ShapeDtypeStruct((NTOK, HIDDEN), jnp.float32),
        mesh=mesh,
        compiler_params=pltpu.CompilerParams(needs_layout_passes=False),
        scratch_types=[
            pltpu.VMEM((SCTOK,), jnp.int32),
            pltpu.VMEM((SCTOK + L,), jnp.int32),
            pltpu.VMEM((C, HIDDEN), jnp.float32),
            pltpu.VMEM((C, HIDDEN), jnp.float32),
            pltpu.VMEM((C, HIDDEN), jnp.float32),
            pltpu.VMEM((C, HIDDEN), jnp.float32),
            pltpu.VMEM((SEQ, HIDDEN), jnp.float32),
            pltpu.VMEM((3, HIDDEN), jnp.float32),
            pltpu.VMEM((HIDDEN,), jnp.float32),
            pltpu.VMEM((HIDDEN,), jnp.float32),
            pltpu.SemaphoreType.DMA,
            pltpu.SemaphoreType.DMA,
        ],
    )
    return f(ids, tts, tok_emb, pos_emb, type_emb, gamma, beta)


def kernel(input_ids, token_type_ids, tok_emb, pos_emb, type_emb, gamma, beta):
    ids = input_ids.reshape(-1).astype(jnp.int32)
    tts = token_type_ids.reshape(-1).astype(jnp.int32)
    out = _run(ids, tts, tok_emb, pos_emb, type_emb, gamma, beta)
    return out.reshape(BATCH, SEQ, HIDDEN)


# speedup vs baseline: 7.5631x; 2.0738x over previous
import functools

import jax
import jax.numpy as jnp
from jax import lax
from jax.experimental import pallas as pl
from jax.experimental.pallas import tpu as pltpu
from jax.experimental.pallas import tpu_sc as plsc

NC, NS, L = 2, 16, 16
NW = NC * NS
BATCH, SEQ, HIDDEN = 1024, 512, 128
NTOK = BATCH * SEQ
TPW = NTOK // NW
C = 64
CPSS = 32
SCTOK = C * CPSS
NSS = TPW // SCTOK
NJ = HIDDEN // L


def _rsqrt16(v):
    i = lax.bitcast_convert_type(v, jnp.int32)
    y = lax.bitcast_convert_type(jnp.int32(0x5F3759DF) - (i >> 1), jnp.float32)
    for _ in range(2):
        y = y * (1.5 - 0.5 * v * y * y)
    return y


def _body(ids_hbm, tt_hbm, tok_hbm, pos_hbm, type_hbm, gam_hbm, bet_hbm,
          out_hbm, idx_sc, tt_sc, rows0, rows1, out0, out1, pos_v, type_v,
          gam_v, bet_v, sem_g, sem_o):
    rows = (rows0, rows1)
    outs = (out0, out1)
    wid = lax.axis_index("s") * NC + lax.axis_index("c")

    pltpu.sync_copy(pos_hbm, pos_v)
    pltpu.sync_copy(type_hbm, type_v)
    pltpu.sync_copy(gam_hbm, gam_v)
    pltpu.sync_copy(bet_hbm, bet_v)

    def gather_start(k, buf):
        pltpu.async_copy(tok_hbm.at[idx_sc.at[pl.ds(k * C, C)]], buf, sem_g)

    def gather_wait(buf):
        pltpu.make_async_copy(
            tok_hbm.at[idx_sc.at[pl.ds(0, C)]], buf, sem_g).wait()

    def out_wait(buf):
        pltpu.make_async_copy(buf, out_hbm.at[pl.ds(0, C)], sem_o).wait()

    def compute(p0, toff, rbuf, obuf):
        @plsc.parallel_loop(0, C, unroll=8)
        def _tok(i):
            tt = tt_sc[pl.ds(toff + i, L)][0]
            p = p0 + i
            xs = []
            for j in range(NJ):
                sl = pl.ds(j * L, L)
                xs.append(rbuf[i, sl] + pos_v[p, sl] + type_v[tt, sl])

            def tree(vs):
                while len(vs) > 1:
                    vs = [a + b for a, b in zip(vs[::2], vs[1::2])]
                return vs[0]

            s = tree(xs)
            sq = tree([x * x for x in xs])
            tot = jnp.sum(s)
            totsq = jnp.sum(sq)
            meanv = lax.broadcast(tot, (L,)) * (1.0 / HIDDEN)
            varv = lax.broadcast(totsq, (L,)) * (1.0 / HIDDEN) \
                - meanv * meanv + 1e-5
            rstd = _rsqrt16(varv)
            for j in range(NJ):
                sl = pl.ds(j * L, L)
                obuf[i, sl] = (xs[j] - meanv) * rstd * gam_v[sl] + bet_v[sl]

    @pl.loop(0, NSS)
    def _ss(s):
        ids_off = wid * TPW + s * SCTOK
        pltpu.sync_copy(ids_hbm.at[pl.ds(ids_off, SCTOK)], idx_sc)
        pltpu.sync_copy(tt_hbm.at[pl.ds(ids_off, SCTOK)],
                        tt_sc.at[pl.ds(0, SCTOK)])
        gather_start(0, rows[0])

        @pl.loop(0, CPSS, step=2)
        def _chunk(k0):
            for b in range(2):
                k = k0 + b
                gci = s * CPSS + k
                gather_wait(rows[b])

                @pl.when(k + 1 < CPSS)
                def _():
                    gather_start(k + 1, rows[1 - b])

                @pl.when(gci >= 2)
                def _():
                    out_wait(outs[b])

                p0 = (gci % (SEQ // C)) * C
                compute(p0, k * C, rows[b], outs[b])
                pltpu.async_copy(
                    outs[b], out_hbm.at[pl.ds(wid * TPW + gci * C, C)],
                    sem_o)

    out_wait(outs[0])
    out_wait(outs[1])


@jax.jit
def _run(ids, tts, tok_emb, pos_emb, type_emb, gamma, beta):
    mesh = plsc.VectorSubcoreMesh(core_axis_name="c", subcore_axis_name="s",
                                  num_cores=NC, num_subcores=NS)
    f = pl.kernel(
        _body,
        out_type=jax.ShapeDtypeStruct((NTOK, HIDDEN), jnp.float32),
        mesh=mesh,
        compiler_params=pltpu.CompilerParams(needs_layout_passes=False),
        scratch_types=[
            pltpu.VMEM((SCTOK,), jnp.int32),
            pltpu.VMEM((SCTOK + L,), jnp.int32),
            pltpu.VMEM((C, HIDDEN), jnp.float32),
            pltpu.VMEM((C, HIDDEN), jnp.float32),
            pltpu.VMEM((C, HIDDEN), jnp.float32),
            pltpu.VMEM((C, HIDDEN), jnp.float32),
            pltpu.VMEM((SEQ, HIDDEN), jnp.float32),
            pltpu.VMEM((3, HIDDEN), jnp.float32),
            pltpu.VMEM((HIDDEN,), jnp.float32),
            pltpu.VMEM((HIDDEN,), jnp.float32),
            pltpu.SemaphoreType.DMA,
            pltpu.SemaphoreType.DMA,
        ],
    )
    return f(ids, tts, tok_emb, pos_emb, type_emb, gamma, beta)


def kernel(input_ids, token_type_ids, tok_emb, pos_emb, type_emb, gamma, beta):
    ids = input_ids.reshape(-1).astype(jnp.int32)
    tts = token_type_ids.reshape(-1).astype(jnp.int32)
    out = _run(ids, tts, tok_emb, pos_emb, type_emb, gamma, beta)
    return out.reshape(BATCH, SEQ, HIDDEN)
